# baseline (speedup 1.0000x reference)
import jax
import jax.numpy as jnp
from jax.experimental import pallas as pl
from jax.experimental.pallas import tpu as pltpu

_P = 28
_H = 112
_W = 112
_C = 128
_D = 4
_NP = 56
_THRESH = 0.9


def _msd_body(xe_ref, xo_ref, wi_ref, bi_ref, wb_ref, bb_ref, cw_ref,
              cb_ref, o_ref, ea, oa, eb, ob):
    nt = _NP // _P
    m = _P * _W

    for f in (ea, oa, eb, ob):
        f[:, 0:8, :] = jnp.zeros((57, 8, _C), jnp.float32)
        f[:, 120:128, :] = jnp.zeros((57, 8, _C), jnp.float32)
    for f in (ea, eb):
        f[56, :, :] = jnp.zeros((128, _C), jnp.float32)
    for f in (oa, ob):
        f[0, :, :] = jnp.zeros((128, _C), jnp.float32)

    ncol = xe_ref.shape[3]
    for rt in range(nt):
        p0 = rt * _P
        for x_ref, dstb, off in ((xe_ref, ea, 0), (xo_ref, oa, 1)):
            xs = x_ref[0, pl.ds(p0, _P), pl.ds(8, _W), :]
            y = jnp.dot(xs.reshape(m, ncol), wi_ref[:, :],
                        preferred_element_type=jnp.float32) + bi_ref[0]
            dstb[pl.ds(p0 + off, _P), pl.ds(8, _W), :] = (
                y.reshape(_P, _W, _C))

    bufs = ((ea, oa), (eb, ob))
    out = jnp.zeros((_C,), jnp.float32)
    exited = jnp.zeros((), jnp.bool_)
    logits = None
    for d in range(_D):
        se, so = bufs[d % 2]
        de, do = bufs[(d + 1) % 2]
        psum = jnp.zeros((_C,), jnp.float32)
        for rt in range(nt):
            p0 = rt * _P
            ble = se[pl.ds(p0, _P + 1), :, :]
            blo = so[pl.ds(p0, _P + 1), :, :]
            she = [ble[:, 7 + kx:119 + kx, :] for kx in range(3)]
            sho = [blo[:, 7 + kx:119 + kx, :] for kx in range(3)]
            parts = []
            for rr, sh in ((0, sho), (1, she), (2, sho), (3, she)):
                lo = rr // 2
                for kx in range(3):
                    parts.append(sh[kx][lo:lo + _P].reshape(m, _C)
                                 .astype(jnp.bfloat16))
            xcat = jnp.concatenate(parts, axis=1)
            acc = jnp.dot(xcat, wb_ref[d],
                          preferred_element_type=jnp.float32)
            a = jnp.maximum(acc + bb_ref[d], 0.0)
            de[pl.ds(p0, _P), pl.ds(8, _W), :] = (
                a[:, :_C].reshape(_P, _W, _C))
            do[pl.ds(p0 + 1, _P), pl.ds(8, _W), :] = (
                a[:, _C:].reshape(_P, _W, _C))
            s2 = jnp.sum(a, axis=0)
            psum = psum + s2[:_C] + s2[_C:]
        pooled = psum * (1.0 / float(_H * _W))
        logits = (jnp.dot(pooled[None, :], cw_ref[d],
                          preferred_element_type=jnp.float32)[0]
                  + cb_ref[d])
        mx = jnp.max(logits)
        conf = 1.0 / jnp.sum(jnp.exp(logits - mx))
        take = jnp.logical_and(jnp.logical_not(exited), conf >= _THRESH)
        out = jnp.where(take, logits, out)
        exited = jnp.logical_or(exited, take)
    out = jnp.where(exited, out, logits)
    o_ref[0, 0, :] = out


def kernel(x, init_w, init_b, block_w, block_b, cls_w, cls_b):
    b = x.shape[0]
    cin = x.shape[1]
    nc = cls_w.shape[1]

    xt = jnp.transpose(x, (0, 2, 3, 1))
    xim = jnp.pad(xt, ((0, 0), (1, 1), (1, 1), (0, 0)))
    x_col = jnp.concatenate(
        [xim[:, ky:ky + _H, kx:kx + _W, :]
         for ky in range(3) for kx in range(3)], axis=3)
    x_col = jnp.pad(x_col, ((0, 0), (0, 0), (8, 8), (0, 0)))
    x_even = x_col[:, 0::2]
    x_odd = x_col[:, 1::2]

    wi = jnp.transpose(init_w[2], (2, 3, 1, 0)).reshape(9 * cin, _C)
    bi = init_b[2].reshape(1, _C)
    wt = jnp.transpose(block_w[:, 2], (0, 3, 4, 2, 1))
    wb = jnp.zeros((_D, 4, 3, _C, 2 * _C), block_w.dtype)
    wb = wb.at[:, :3, :, :, :_C].set(wt)
    wb = wb.at[:, 1:, :, :, _C:].set(wt)
    wb = wb.reshape(_D, 12 * _C, 2 * _C).astype(jnp.bfloat16)
    bb = jnp.tile(block_b[:, 2], (1, 2))
    cw = jnp.pad(jnp.transpose(cls_w, (0, 2, 1)),
                 ((0, 0), (0, 0), (0, _C - nc)))
    cb = jnp.pad(cls_b, ((0, 0), (0, _C - nc)), constant_values=-1e30)

    feat = lambda: pltpu.VMEM((57, 128, _C), jnp.float32)
    out_pad = pl.pallas_call(
        _msd_body,
        grid=(b,),
        in_specs=[
            pl.BlockSpec((1, _NP, _C, 9 * cin), lambda i: (i, 0, 0, 0)),
            pl.BlockSpec((1, _NP, _C, 9 * cin), lambda i: (i, 0, 0, 0)),
            pl.BlockSpec((9 * cin, _C), lambda i: (0, 0)),
            pl.BlockSpec((1, _C), lambda i: (0, 0)),
            pl.BlockSpec((_D, 12 * _C, 2 * _C), lambda i: (0, 0, 0)),
            pl.BlockSpec((_D, 2 * _C), lambda i: (0, 0)),
            pl.BlockSpec((_D, _C, _C), lambda i: (0, 0, 0)),
            pl.BlockSpec((_D, _C), lambda i: (0, 0)),
        ],
        out_specs=pl.BlockSpec((1, 1, _C), lambda i: (i, 0, 0)),
        out_shape=jax.ShapeDtypeStruct((b, 1, _C), jnp.float32),
        scratch_shapes=[feat() for _ in range(4)],
        compiler_params=pltpu.CompilerParams(
            dimension_semantics=("parallel",)),
    )(x_even, x_odd, wi, bi, wb, bb, cw, cb)
    return out_pad[:, 0, :nc]


# speedup vs baseline: 1.0015x; 1.0015x over previous
import jax
import jax.numpy as jnp
from jax.experimental import pallas as pl
from jax.experimental.pallas import tpu as pltpu

_P = 14
_H = 112
_W = 112
_C = 128
_D = 4
_NP = 56
_THRESH = 0.9


def _msd_body(xe_ref, xo_ref, wi_ref, bi_ref, wb_ref, bb_ref, cw_ref,
              cb_ref, o_ref, ea, oa, eb, ob):
    nt = _NP // _P
    m = _P * _W

    for f in (ea, oa, eb, ob):
        f[:, 0:8, :] = jnp.zeros((57, 8, _C), jnp.float32)
        f[:, 120:128, :] = jnp.zeros((57, 8, _C), jnp.float32)
    for f in (ea, eb):
        f[56, :, :] = jnp.zeros((128, _C), jnp.float32)
    for f in (oa, ob):
        f[0, :, :] = jnp.zeros((128, _C), jnp.float32)

    ncol = xe_ref.shape[3]
    for rt in range(nt):
        p0 = rt * _P
        for x_ref, dstb, off in ((xe_ref, ea, 0), (xo_ref, oa, 1)):
            xs = x_ref[0, pl.ds(p0, _P), pl.ds(8, _W), :]
            y = jnp.dot(xs.reshape(m, ncol), wi_ref[:, :],
                        preferred_element_type=jnp.float32) + bi_ref[0]
            dstb[pl.ds(p0 + off, _P), pl.ds(8, _W), :] = (
                y.reshape(_P, _W, _C))

    bufs = ((ea, oa), (eb, ob))
    out = jnp.zeros((_C,), jnp.float32)
    exited = jnp.zeros((), jnp.bool_)
    logits = None
    for d in range(_D):
        se, so = bufs[d % 2]
        de, do = bufs[(d + 1) % 2]
        psum = jnp.zeros((_C,), jnp.float32)
        for rt in range(nt):
            p0 = rt * _P
            ble = se[pl.ds(p0, _P + 1), :, :]
            blo = so[pl.ds(p0, _P + 1), :, :]
            she = [ble[:, 7 + kx:119 + kx, :] for kx in range(3)]
            sho = [blo[:, 7 + kx:119 + kx, :] for kx in range(3)]
            parts = []
            for rr, sh in ((0, sho), (1, she), (2, sho), (3, she)):
                lo = rr // 2
                for kx in range(3):
                    parts.append(sh[kx][lo:lo + _P].reshape(m, _C)
                                 .astype(jnp.bfloat16))
            acc = jnp.zeros((m, 2 * _C), jnp.float32)
            for i in range(6):
                pair = jnp.concatenate(parts[2 * i:2 * i + 2], axis=1)
                acc = acc + jnp.dot(
                    pair, wb_ref[d, pl.ds(2 * i * _C, 2 * _C), :],
                    preferred_element_type=jnp.float32)
            a = jnp.maximum(acc + bb_ref[d], 0.0)
            de[pl.ds(p0, _P), pl.ds(8, _W), :] = (
                a[:, :_C].reshape(_P, _W, _C))
            do[pl.ds(p0 + 1, _P), pl.ds(8, _W), :] = (
                a[:, _C:].reshape(_P, _W, _C))
            s2 = jnp.sum(a, axis=0)
            psum = psum + s2[:_C] + s2[_C:]
        pooled = psum * (1.0 / float(_H * _W))
        logits = (jnp.dot(pooled[None, :], cw_ref[d],
                          preferred_element_type=jnp.float32)[0]
                  + cb_ref[d])
        mx = jnp.max(logits)
        conf = 1.0 / jnp.sum(jnp.exp(logits - mx))
        take = jnp.logical_and(jnp.logical_not(exited), conf >= _THRESH)
        out = jnp.where(take, logits, out)
        exited = jnp.logical_or(exited, take)
    out = jnp.where(exited, out, logits)
    o_ref[0, 0, :] = out


def kernel(x, init_w, init_b, block_w, block_b, cls_w, cls_b):
    b = x.shape[0]
    cin = x.shape[1]
    nc = cls_w.shape[1]

    xt = jnp.transpose(x, (0, 2, 3, 1))
    xim = jnp.pad(xt, ((0, 0), (1, 1), (1, 1), (0, 0)))
    x_col = jnp.concatenate(
        [xim[:, ky:ky + _H, kx:kx + _W, :]
         for ky in range(3) for kx in range(3)], axis=3)
    x_col = jnp.pad(x_col, ((0, 0), (0, 0), (8, 8), (0, 0)))
    x_even = x_col[:, 0::2]
    x_odd = x_col[:, 1::2]

    wi = jnp.transpose(init_w[2], (2, 3, 1, 0)).reshape(9 * cin, _C)
    bi = init_b[2].reshape(1, _C)
    wt = jnp.transpose(block_w[:, 2], (0, 3, 4, 2, 1))
    wb = jnp.zeros((_D, 4, 3, _C, 2 * _C), block_w.dtype)
    wb = wb.at[:, :3, :, :, :_C].set(wt)
    wb = wb.at[:, 1:, :, :, _C:].set(wt)
    wb = wb.reshape(_D, 12 * _C, 2 * _C).astype(jnp.bfloat16)
    bb = jnp.tile(block_b[:, 2], (1, 2))
    cw = jnp.pad(jnp.transpose(cls_w, (0, 2, 1)),
                 ((0, 0), (0, 0), (0, _C - nc)))
    cb = jnp.pad(cls_b, ((0, 0), (0, _C - nc)), constant_values=-1e30)

    feat = lambda: pltpu.VMEM((57, 128, _C), jnp.float32)
    out_pad = pl.pallas_call(
        _msd_body,
        grid=(b,),
        in_specs=[
            pl.BlockSpec((1, _NP, _C, 9 * cin), lambda i: (i, 0, 0, 0)),
            pl.BlockSpec((1, _NP, _C, 9 * cin), lambda i: (i, 0, 0, 0)),
            pl.BlockSpec((9 * cin, _C), lambda i: (0, 0)),
            pl.BlockSpec((1, _C), lambda i: (0, 0)),
            pl.BlockSpec((_D, 12 * _C, 2 * _C), lambda i: (0, 0, 0)),
            pl.BlockSpec((_D, 2 * _C), lambda i: (0, 0)),
            pl.BlockSpec((_D, _C, _C), lambda i: (0, 0, 0)),
            pl.BlockSpec((_D, _C), lambda i: (0, 0)),
        ],
        out_specs=pl.BlockSpec((1, 1, _C), lambda i: (i, 0, 0)),
        out_shape=jax.ShapeDtypeStruct((b, 1, _C), jnp.float32),
        scratch_shapes=[feat() for _ in range(4)],
        compiler_params=pltpu.CompilerParams(
            dimension_semantics=("parallel",)),
    )(x_even, x_odd, wi, bi, wb, bb, cw, cb)
    return out_pad[:, 0, :nc]


# speedup vs baseline: 1.0238x; 1.0222x over previous
import jax
import jax.numpy as jnp
from jax.experimental import pallas as pl
from jax.experimental.pallas import tpu as pltpu

_P = 14
_H = 112
_W = 112
_C = 128
_D = 4
_NP = 56
_THRESH = 0.9


def _msd_body(xe_ref, xo_ref, wi_ref, bi_ref, wb_ref, bb_ref, cw_ref,
              cb_ref, o_ref, ea, oa, eb, ob):
    nt = _NP // _P
    m = _P * _W

    for f in (ea, oa, eb, ob):
        f[:, 0:8, :] = jnp.zeros((57, 8, _C), jnp.float32)
        f[:, 120:128, :] = jnp.zeros((57, 8, _C), jnp.float32)
    for f in (ea, eb):
        f[56, :, :] = jnp.zeros((128, _C), jnp.float32)
    for f in (oa, ob):
        f[0, :, :] = jnp.zeros((128, _C), jnp.float32)

    ncol = xe_ref.shape[3]
    for rt in range(nt):
        p0 = rt * _P
        for x_ref, dstb, off in ((xe_ref, ea, 0), (xo_ref, oa, 1)):
            xs = x_ref[0, pl.ds(p0, _P), pl.ds(8, _W), :]
            y = jnp.dot(xs.reshape(m, ncol), wi_ref[:, :],
                        preferred_element_type=jnp.float32) + bi_ref[0]
            dstb[pl.ds(p0 + off, _P), pl.ds(8, _W), :] = (
                y.reshape(_P, _W, _C))

    bufs = ((ea, oa), (eb, ob))
    out = jnp.zeros((_C,), jnp.float32)
    exited = jnp.zeros((), jnp.bool_)
    logits = None
    for d in range(_D):
        se, so = bufs[d % 2]
        de, do = bufs[(d + 1) % 2]
        psum = jnp.zeros((_C,), jnp.float32)
        for rt in range(nt):
            p0 = rt * _P
            ble = se[pl.ds(p0, _P + 1), :, :]
            blo = so[pl.ds(p0, _P + 1), :, :]
            she = [ble[:, 7 + kx:119 + kx, :] for kx in range(3)]
            sho = [blo[:, 7 + kx:119 + kx, :] for kx in range(3)]
            parts = []
            for rr, sh in ((0, sho), (1, she), (2, sho), (3, she)):
                lo = rr // 2
                for kx in range(3):
                    parts.append(sh[kx][lo:lo + _P].reshape(m, _C)
                                 .astype(jnp.bfloat16))
            xcat = jnp.concatenate(parts, axis=1)
            acc = jnp.dot(xcat, wb_ref[d],
                          preferred_element_type=jnp.float32)
            a = jnp.maximum(acc + bb_ref[d], 0.0)
            de[pl.ds(p0, _P), pl.ds(8, _W), :] = (
                a[:, :_C].reshape(_P, _W, _C))
            do[pl.ds(p0 + 1, _P), pl.ds(8, _W), :] = (
                a[:, _C:].reshape(_P, _W, _C))
            s2 = jnp.sum(a, axis=0)
            psum = psum + s2[:_C] + s2[_C:]
        pooled = psum * (1.0 / float(_H * _W))
        logits = (jnp.dot(pooled[None, :], cw_ref[d],
                          preferred_element_type=jnp.float32)[0]
                  + cb_ref[d])
        mx = jnp.max(logits)
        conf = 1.0 / jnp.sum(jnp.exp(logits - mx))
        take = jnp.logical_and(jnp.logical_not(exited), conf >= _THRESH)
        out = jnp.where(take, logits, out)
        exited = jnp.logical_or(exited, take)
    out = jnp.where(exited, out, logits)
    o_ref[0, 0, :] = out


def kernel(x, init_w, init_b, block_w, block_b, cls_w, cls_b):
    b = x.shape[0]
    cin = x.shape[1]
    nc = cls_w.shape[1]

    xt = jnp.transpose(x, (0, 2, 3, 1))
    xim = jnp.pad(xt, ((0, 0), (1, 1), (1, 1), (0, 0)))
    x_col = jnp.concatenate(
        [xim[:, ky:ky + _H, kx:kx + _W, :]
         for ky in range(3) for kx in range(3)], axis=3)
    x_col = jnp.pad(x_col, ((0, 0), (0, 0), (8, 8), (0, 0)))
    x_col = x_col.astype(jnp.bfloat16)
    x_even = x_col[:, 0::2]
    x_odd = x_col[:, 1::2]

    wi = jnp.transpose(init_w[2], (2, 3, 1, 0)).reshape(9 * cin, _C)
    wi = wi.astype(jnp.bfloat16)
    bi = init_b[2].reshape(1, _C)
    wt = jnp.transpose(block_w[:, 2], (0, 3, 4, 2, 1))
    wb = jnp.zeros((_D, 4, 3, _C, 2 * _C), block_w.dtype)
    wb = wb.at[:, :3, :, :, :_C].set(wt)
    wb = wb.at[:, 1:, :, :, _C:].set(wt)
    wb = wb.reshape(_D, 12 * _C, 2 * _C).astype(jnp.bfloat16)
    bb = jnp.tile(block_b[:, 2], (1, 2))
    cw = jnp.pad(jnp.transpose(cls_w, (0, 2, 1)),
                 ((0, 0), (0, 0), (0, _C - nc)))
    cb = jnp.pad(cls_b, ((0, 0), (0, _C - nc)), constant_values=-1e30)

    feat = lambda: pltpu.VMEM((57, 128, _C), jnp.float32)
    out_pad = pl.pallas_call(
        _msd_body,
        grid=(b,),
        in_specs=[
            pl.BlockSpec((1, _NP, _C, 9 * cin), lambda i: (i, 0, 0, 0)),
            pl.BlockSpec((1, _NP, _C, 9 * cin), lambda i: (i, 0, 0, 0)),
            pl.BlockSpec((9 * cin, _C), lambda i: (0, 0)),
            pl.BlockSpec((1, _C), lambda i: (0, 0)),
            pl.BlockSpec((_D, 12 * _C, 2 * _C), lambda i: (0, 0, 0)),
            pl.BlockSpec((_D, 2 * _C), lambda i: (0, 0)),
            pl.BlockSpec((_D, _C, _C), lambda i: (0, 0, 0)),
            pl.BlockSpec((_D, _C), lambda i: (0, 0)),
        ],
        out_specs=pl.BlockSpec((1, 1, _C), lambda i: (i, 0, 0)),
        out_shape=jax.ShapeDtypeStruct((b, 1, _C), jnp.float32),
        scratch_shapes=[feat() for _ in range(4)],
        compiler_params=pltpu.CompilerParams(
            dimension_semantics=("parallel",)),
    )(x_even, x_odd, wi, bi, wb, bb, cw, cb)
    return out_pad[:, 0, :nc]
